# Initial kernel scaffold; baseline (speedup 1.0000x reference)
import jax
import jax.numpy as jnp
from jax.experimental import pallas as pl


def kernel(fV, seg, byx, bb, nV, pixel_mask_token, mix_logit):
    raise NotImplementedError("write your pallas kernel here")


# speedup vs baseline: 15.1973x; 15.1973x over previous
import jax
import jax.numpy as jnp
from jax import lax
from jax.experimental import pallas as pl
from jax.experimental.pallas import tpu as pltpu
from jax.experimental.pallas import tpu_sc as plsc

H = 1024
W = 1024
PS = 16
NV = 4096
N = H * W
NC = 2
NS = 16
PIX_PER_TILE = N // NS
CHUNK = 2048
SEGS_PER_TILE = NV // (NC * NS)
M_HI = -65536


def _body(ta_hbm, tb_hbm, bb_hbm, gb_hbm, tok_hbm, mixv_hbm, out_hbm,
          zeros_v, ones_v, segblk_v, idxh_v, bb_v, gb_v, tok_v, mix_v,
          gidx_v, ga_v, gb2_v, gr_v, red_v, outb_v, grid_sh, sem):
    sid = lax.axis_index("s")
    cid = lax.axis_index("c")

    zero16 = jnp.zeros((16,), jnp.float32)
    for k in range(128):
        zeros_v[pl.ds(k * 16, 16)] = zero16
    one16 = jnp.ones((16,), jnp.float32)
    for k in range(8):
        ones_v[pl.ds(k * 16, 16)] = one16
    red_v[pl.ds(0, 16)] = zero16
    red_v[pl.ds(32, 16)] = zero16

    def zbody(q, carry):
        pltpu.sync_copy(zeros_v,
                        grid_sh.at[pl.ds(sid * PIX_PER_TILE + q * CHUNK, CHUNK)])
        return carry
    lax.fori_loop(0, PIX_PER_TILE // CHUNK, zbody, 0)
    plsc.subcore_barrier()

    iot = lax.iota(jnp.int32, 16)

    def chunk_body(ci, carry):
        base = sid * PIX_PER_TILE + ci * CHUNK
        pltpu.sync_copy(tb_hbm.at[pl.ds(base, CHUNK)], segblk_v)
        for k in range(CHUNK // 16):
            s = segblk_v[pl.ds(k * 16, 16)] & 0xFFFF
            p = iot + (base + k * 16)
            t = ((p >> 16) << 4) + ((p >> 6) & 15)
            idxh_v[k // 8, pl.ds((k % 8) * 16, 16)] = (s << 8) + t
        for q in range(16):
            pltpu.sync_copy(ones_v, grid_sh.at[idxh_v.at[q]], add=True)
        return carry
    lax.fori_loop(0, PIX_PER_TILE // CHUNK, chunk_body, 0)
    plsc.subcore_barrier()

    wid = cid * NS + sid
    v0 = wid * SEGS_PER_TILE
    pltpu.sync_copy(bb_hbm.at[pl.ds(v0 * 64, SEGS_PER_TILE * 64)], bb_v)
    pltpu.sync_copy(gb_hbm, gb_v)
    pltpu.sync_copy(tok_hbm, tok_v)
    pltpu.sync_copy(mixv_hbm, mix_v)
    mix_b = mix_v[...]
    omix_b = 1.0 - mix_b
    gbase = gb_v[pl.ds(0, 16)]

    def seg_body(tloc, carry):
        v = v0 + tloc
        off = tloc * 64
        ymin = bb_v[pl.ds(off, 16)]
        xmin = bb_v[pl.ds(off + 16, 16)]
        ymax = bb_v[pl.ds(off + 32, 16)]
        xmax = bb_v[pl.ds(off + 48, 16)]
        dh = ymax - ymin
        wpos = gbase * (xmax - xmin) + xmin
        wf = jnp.minimum(jnp.maximum(wpos.astype(jnp.int32), 0), W - 1)
        wc = jnp.minimum(wf + 1, W - 1)
        uw = wpos - wf.astype(jnp.float32)
        lw = 1.0 - uw

        def hrow(i):
            g_i = gb_v[pl.ds(16 + i * 16, 16)]
            hpos = g_i * dh + ymin
            hf = jnp.minimum(jnp.maximum(hpos.astype(jnp.int32), 0), H - 1)
            hc = jnp.minimum(hf + 1, H - 1)
            uh = hpos - hf.astype(jnp.float32)
            return hf, hc, uh

        for i in range(16):
            hf, hc, _ = hrow(i)
            rf_b = hf << 10
            rc_b = hc << 10
            for corner, idx in ((0, rf_b + wf), (1, rf_b + wc),
                                (2, rc_b + wf), (3, rc_b + wc)):
                e = corner * 256 + i * 16
                gidx_v[e // 128, pl.ds(e % 128, 16)] = idx
        copies = []
        for q in range(8):
            copies.append(pltpu.async_copy(
                ta_hbm.at[gidx_v.at[q]], ga_v.at[pl.ds(q * 128, 128)], sem))
            copies.append(pltpu.async_copy(
                tb_hbm.at[gidx_v.at[q]], gb2_v.at[pl.ds(q * 128, 128)], sem))
        for c in copies:
            c.wait()

        vi_b = jnp.broadcast_to(v, (16,))
        for i in range(16):
            _, _, uhb = hrow(i)
            lhb = 1.0 - uhb
            wff = lhb * lw
            wfc = lhb * uw
            wcf = uhb * lw
            wcc = uhb * uw
            fs = []
            m = jnp.zeros((16,), jnp.float32)
            for corner, wgt in ((0, wff), (1, wfc), (2, wcf), (3, wcc)):
                base = corner * 256 + i * 16
                wa = ga_v[pl.ds(base, 16)]
                wb = gb2_v[pl.ds(base, 16)]
                f0 = lax.bitcast_convert_type(wa & M_HI, jnp.float32)
                f1 = lax.bitcast_convert_type(wa << 16, jnp.float32)
                f2 = lax.bitcast_convert_type(wb & M_HI, jnp.float32)
                sg = wb & 0xFFFF
                fs.append((f0, f1, f2))
                m = m + jnp.where(sg == vi_b, wgt, 0.0)
            a_b = m + mix_b * (1.0 - m)
            t_b = omix_b * (1.0 - m)
            for c in range(3):
                bil = (fs[0][c] * wff + fs[1][c] * wfc
                       + fs[2][c] * wcf + fs[3][c] * wcc)
                tokci = tok_v[pl.ds(c * 256 + i * 16, 16)]
                outb_v[pl.ds(c * 256 + i * 16, 16)] = bil * a_b + t_b * tokci

        pltpu.sync_copy(grid_sh.at[pl.ds(v * 256, 256)], gr_v)
        acc = gr_v[pl.ds(0, 16)]
        for k in range(1, 16):
            acc = acc + gr_v[pl.ds(k * 16, 16)]
        for d in (8, 4, 2, 1):
            red_v[pl.ds(16, 16)] = acc
            plus = red_v[pl.ds(16 + d, 16)]
            minus = red_v[pl.ds(16 - d, 16)]
            acc = acc + jnp.where((iot & d) == 0, plus, minus)
        scale_b = 4.0 / acc
        for k in range(16):
            outb_v[pl.ds(768 + k * 16, 16)] = gr_v[pl.ds(k * 16, 16)] * scale_b
        pltpu.sync_copy(outb_v, out_hbm.at[v])
        return carry
    lax.fori_loop(0, SEGS_PER_TILE, seg_body, 0)


_sc_call = pl.kernel(
    _body,
    out_type=jax.ShapeDtypeStruct((NV, 4 * PS * PS), jnp.float32),
    mesh=plsc.VectorSubcoreMesh(core_axis_name="c", subcore_axis_name="s"),
    scratch_types=[
        pltpu.VMEM((CHUNK,), jnp.float32),
        pltpu.VMEM((128,), jnp.float32),
        pltpu.VMEM((CHUNK,), jnp.int32),
        pltpu.VMEM((16, 128), jnp.int32),
        pltpu.VMEM((SEGS_PER_TILE * 64,), jnp.float32),
        pltpu.VMEM((16 + 256,), jnp.float32),
        pltpu.VMEM((3 * PS * PS,), jnp.float32),
        pltpu.VMEM((16,), jnp.float32),
        pltpu.VMEM((8, 128), jnp.int32),
        pltpu.VMEM((1024,), jnp.int32),
        pltpu.VMEM((1024,), jnp.int32),
        pltpu.VMEM((256,), jnp.float32),
        pltpu.VMEM((48,), jnp.float32),
        pltpu.VMEM((4 * PS * PS,), jnp.float32),
        pltpu.VMEM_SHARED((NV * PS * PS,), jnp.float32),
        pltpu.SemaphoreType.DMA,
    ],
)


def kernel(fV, seg, byx, bb, nV, pixel_mask_token, mix_logit):
    seg_u = seg.reshape(-1).astype(jnp.uint32)
    fb = fV.astype(jnp.bfloat16)
    u = lax.bitcast_convert_type(fb, jnp.uint16).astype(jnp.uint32)
    word_a = (u[:, 0] << 16) | u[:, 1]
    word_b = (u[:, 2] << 16) | seg_u
    tab_a = lax.bitcast_convert_type(word_a, jnp.int32)
    tab_b = lax.bitcast_convert_type(word_b, jnp.int32)
    bb_bc = jnp.repeat(bb.T.reshape(-1), 16)
    grid_base = jnp.linspace(0.0, 1.0, PS)
    gb_all = jnp.concatenate([grid_base, jnp.repeat(grid_base, 16)])
    mix = jax.nn.sigmoid(mix_logit)[0]
    mixv = jnp.full((16,), mix, jnp.float32)
    tok = pixel_mask_token.reshape(3 * PS * PS)
    out = _sc_call(tab_a, tab_b, bb_bc, gb_all, tok, mixv)
    return out.reshape(NV, 4, PS, PS)


# speedup vs baseline: 17.0600x; 1.1226x over previous
import jax
import jax.numpy as jnp
from jax import lax
from jax.experimental import pallas as pl
from jax.experimental.pallas import tpu as pltpu
from jax.experimental.pallas import tpu_sc as plsc

H = 1024
W = 1024
PS = 16
NV = 4096
N = H * W
NC = 2
NS = 16
PIX_PER_TILE = N // NS
CHUNK = 2048
NCHUNK = PIX_PER_TILE // CHUNK
SEGS_PER_TILE = NV // (NC * NS)
M_HI = -65536


def _body(ta_hbm, tb_hbm, bb_hbm, gb_hbm, tok_hbm, mixv_hbm, out_hbm,
          zeros_v, ones_v, segblk_v, idxh_v, bb_v, gb_v, tok_v, mix_v,
          gidx_v, ga_v, gb2_v, gr_v, red_v, outb_v, grid_sh,
          sem_g, sem_sc, sem_ld):
    sid = lax.axis_index("s")
    cid = lax.axis_index("c")

    zero16 = jnp.zeros((16,), jnp.float32)
    for k in range(128):
        zeros_v[pl.ds(k * 16, 16)] = zero16
    one16 = jnp.ones((16,), jnp.float32)
    for k in range(8):
        ones_v[pl.ds(k * 16, 16)] = one16
    red_v[pl.ds(0, 16)] = zero16
    red_v[pl.ds(32, 16)] = zero16

    def zbody(q, carry):
        pltpu.sync_copy(zeros_v,
                        grid_sh.at[pl.ds(sid * PIX_PER_TILE + q * CHUNK, CHUNK)])
        return carry
    lax.fori_loop(0, NCHUNK, zbody, 0)

    wid = cid * NS + sid
    v0 = wid * SEGS_PER_TILE
    pltpu.sync_copy(bb_hbm.at[pl.ds(v0 * 64, SEGS_PER_TILE * 64)], bb_v)
    pltpu.sync_copy(gb_hbm, gb_v)
    pltpu.sync_copy(tok_hbm, tok_v)
    pltpu.sync_copy(mixv_hbm, mix_v)
    plsc.subcore_barrier()

    iot = lax.iota(jnp.int32, 16)

    def chunk_body(ci, carry):
        base = sid * PIX_PER_TILE + ci * CHUNK
        pltpu.sync_copy(tb_hbm.at[pl.ds(base, CHUNK)],
                        segblk_v.at[pl.ds(0, CHUNK)])
        for k in range(CHUNK // 16):
            s = segblk_v[pl.ds(k * 16, 16)] & 0xFFFF
            pp = iot + (base + k * 16)
            t = ((pp >> 16) << 4) + ((pp >> 6) & 15)
            idxh_v[k // 8, pl.ds((k % 8) * 16, 16)] = (s << 8) + t
        for q in range(16):
            pltpu.sync_copy(ones_v, grid_sh.at[idxh_v.at[q]], add=True)
        return carry
    lax.fori_loop(0, NCHUNK, chunk_body, 0)
    plsc.subcore_barrier()

    mix_b = mix_v[...]
    omix_b = 1.0 - mix_v[...]
    gbase = gb_v[pl.ds(0, 16)]

    def bbox(tloc):
        off = tloc * 64
        ymin = bb_v[pl.ds(off, 16)]
        xmin = bb_v[pl.ds(off + 16, 16)]
        ymax = bb_v[pl.ds(off + 32, 16)]
        xmax = bb_v[pl.ds(off + 48, 16)]
        return ymin, xmin, ymax, xmax

    def wrow(xmin, xmax):
        wpos = gbase * (xmax - xmin) + xmin
        wf = jnp.minimum(jnp.maximum(wpos.astype(jnp.int32), 0), W - 1)
        wc = jnp.minimum(wf + 1, W - 1)
        uw = wpos - wf.astype(jnp.float32)
        return wf, wc, uw

    def hrow(i, ymin, dh):
        g_i = gb_v[pl.ds(16 + i * 16, 16)]
        hpos = g_i * dh + ymin
        hf = jnp.minimum(jnp.maximum(hpos.astype(jnp.int32), 0), H - 1)
        hc = jnp.minimum(hf + 1, H - 1)
        uh = hpos - hf.astype(jnp.float32)
        return hf, hc, uh

    def build_idx(tloc):
        p = tloc & 1
        ymin, xmin, ymax, xmax = bbox(tloc)
        dh = ymax - ymin
        wf, wc, _ = wrow(xmin, xmax)

        def rows(prow_s):
            for i in range(16):
                hf, hc, _ = hrow(i, ymin, dh)
                rf_b = hf << 10
                rc_b = hc << 10
                for corner, idx in ((0, rf_b + wf), (1, rf_b + wc),
                                    (2, rc_b + wf), (3, rc_b + wc)):
                    e = corner * 256 + i * 16
                    gidx_v[prow_s + e // 128, pl.ds(e % 128, 16)] = idx

        @pl.when(p == 0)
        def _():
            rows(0)

        @pl.when(p == 1)
        def _():
            rows(8)

    def fire(tloc):
        p = tloc & 1
        prow = p * 8
        goff = p * 1024
        for q in range(8):
            pltpu.async_copy(ta_hbm.at[gidx_v.at[prow + q]],
                             ga_v.at[pl.ds(goff + q * 128, 128)], sem_g)
            pltpu.async_copy(tb_hbm.at[gidx_v.at[prow + q]],
                             gb2_v.at[pl.ds(goff + q * 128, 128)], sem_g)

    def g_drain():
        pltpu.make_async_copy(tb_hbm.at[pl.ds(0, 2048)],
                              segblk_v.at[pl.ds(0, 2048)], sem_g).wait()

    def compute_seg(tloc):
        p = tloc & 1
        goff = p * 1024
        v = v0 + tloc
        ymin, xmin, ymax, xmax = bbox(tloc)
        dh = ymax - ymin
        _, _, uw = wrow(xmin, xmax)
        lw = 1.0 - uw
        vi_b = jnp.broadcast_to(v, (16,))
        for i in range(16):
            _, _, uhb = hrow(i, ymin, dh)
            lhb = 1.0 - uhb
            wff = lhb * lw
            wfc = lhb * uw
            wcf = uhb * lw
            wcc = uhb * uw
            fs = []
            m = jnp.zeros((16,), jnp.float32)
            for corner, wgt in ((0, wff), (1, wfc), (2, wcf), (3, wcc)):
                base = goff + corner * 256 + i * 16
                wa = ga_v[pl.ds(base, 16)]
                wb = gb2_v[pl.ds(base, 16)]
                f0 = lax.bitcast_convert_type(wa & M_HI, jnp.float32)
                f1 = lax.bitcast_convert_type(wa << 16, jnp.float32)
                f2 = lax.bitcast_convert_type(wb & M_HI, jnp.float32)
                sg = wb & 0xFFFF
                fs.append((f0, f1, f2))
                m = m + jnp.where(sg == vi_b, wgt, 0.0)
            a_b = m + mix_b * (1.0 - m)
            t_b = omix_b * (1.0 - m)
            for c in range(3):
                bil = (fs[0][c] * wff + fs[1][c] * wfc
                       + fs[2][c] * wcf + fs[3][c] * wcc)
                tokci = tok_v[pl.ds(c * 256 + i * 16, 16)]
                outb_v[pl.ds(c * 256 + i * 16, 16)] = bil * a_b + t_b * tokci

        pltpu.sync_copy(grid_sh.at[pl.ds(v * 256, 256)], gr_v)
        acc = gr_v[pl.ds(0, 16)]
        for k in range(1, 16):
            acc = acc + gr_v[pl.ds(k * 16, 16)]
        for d in (8, 4, 2, 1):
            red_v[pl.ds(16, 16)] = acc
            plus = red_v[pl.ds(16 + d, 16)]
            minus = red_v[pl.ds(16 - d, 16)]
            acc = acc + jnp.where((iot & d) == 0, plus, minus)
        scale_b = 4.0 / acc
        for k in range(16):
            outb_v[pl.ds(768 + k * 16, 16)] = gr_v[pl.ds(k * 16, 16)] * scale_b
        pltpu.sync_copy(outb_v, out_hbm.at[v])

    build_idx(jnp.int32(0))
    fire(jnp.int32(0))

    def seg_body(tloc, carry):
        build_idx(tloc)
        g_drain()
        fire(tloc)
        compute_seg(tloc - 1)
        return carry
    lax.fori_loop(1, SEGS_PER_TILE, seg_body, 0)
    g_drain()
    compute_seg(jnp.int32(SEGS_PER_TILE - 1))


_sc_call = pl.kernel(
    _body,
    out_type=jax.ShapeDtypeStruct((NV, 4 * PS * PS), jnp.float32),
    mesh=plsc.VectorSubcoreMesh(core_axis_name="c", subcore_axis_name="s"),
    scratch_types=[
        pltpu.VMEM((CHUNK,), jnp.float32),
        pltpu.VMEM((128,), jnp.float32),
        pltpu.VMEM((2 * CHUNK,), jnp.int32),
        pltpu.VMEM((32, 128), jnp.int32),
        pltpu.VMEM((SEGS_PER_TILE * 64,), jnp.float32),
        pltpu.VMEM((16 + 256,), jnp.float32),
        pltpu.VMEM((3 * PS * PS,), jnp.float32),
        pltpu.VMEM((16,), jnp.float32),
        pltpu.VMEM((16, 128), jnp.int32),
        pltpu.VMEM((2048,), jnp.int32),
        pltpu.VMEM((2048,), jnp.int32),
        pltpu.VMEM((256,), jnp.float32),
        pltpu.VMEM((48,), jnp.float32),
        pltpu.VMEM((4 * PS * PS,), jnp.float32),
        pltpu.VMEM_SHARED((NV * PS * PS,), jnp.float32),
        pltpu.SemaphoreType.DMA,
        pltpu.SemaphoreType.DMA,
        pltpu.SemaphoreType.DMA,
    ],
)


def kernel(fV, seg, byx, bb, nV, pixel_mask_token, mix_logit):
    seg_u = seg.reshape(-1).astype(jnp.uint32)
    fb = fV.astype(jnp.bfloat16)
    u = lax.bitcast_convert_type(fb, jnp.uint16).astype(jnp.uint32)
    word_a = (u[:, 0] << 16) | u[:, 1]
    word_b = (u[:, 2] << 16) | seg_u
    tab_a = lax.bitcast_convert_type(word_a, jnp.int32)
    tab_b = lax.bitcast_convert_type(word_b, jnp.int32)
    bb_bc = jnp.repeat(bb.T.reshape(-1), 16)
    grid_base = jnp.linspace(0.0, 1.0, PS)
    gb_all = jnp.concatenate([grid_base, jnp.repeat(grid_base, 16)])
    mix = jax.nn.sigmoid(mix_logit)[0]
    mixv = jnp.full((16,), mix, jnp.float32)
    tok = pixel_mask_token.reshape(3 * PS * PS)
    out = _sc_call(tab_a, tab_b, bb_bc, gb_all, tok, mixv)
    return out.reshape(NV, 4, PS, PS)


# speedup vs baseline: 17.0807x; 1.0012x over previous
import jax
import jax.numpy as jnp
from jax import lax
from jax.experimental import pallas as pl
from jax.experimental.pallas import tpu as pltpu
from jax.experimental.pallas import tpu_sc as plsc

H = 1024
W = 1024
PS = 16
NV = 4096
N = H * W
NC = 2
NS = 16
PIX_PER_TILE = N // NS
CHUNK = 2048
NCHUNK = PIX_PER_TILE // CHUNK
SEGS_PER_TILE = NV // (NC * NS)
M_HI = -65536


def _body(ta_hbm, tb_hbm, bb_hbm, gb_hbm, tok_hbm, mixv_hbm, out_hbm,
          zeros_v, ones_v, segblk_v, idxh_v, bb_v, gb_v, tok_v, mix_v,
          gidx_v, ga_v, gb2_v, gr_v, red_v, outb_v, grid_sh,
          sem_g, sem_sc, sem_ld):
    sid = lax.axis_index("s")
    cid = lax.axis_index("c")

    zero16 = jnp.zeros((16,), jnp.float32)
    for k in range(128):
        zeros_v[pl.ds(k * 16, 16)] = zero16
    one16 = jnp.ones((16,), jnp.float32)
    for k in range(8):
        ones_v[pl.ds(k * 16, 16)] = one16
    red_v[pl.ds(0, 16)] = zero16
    red_v[pl.ds(32, 16)] = zero16

    def zbody(q, carry):
        pltpu.sync_copy(zeros_v,
                        grid_sh.at[pl.ds(sid * PIX_PER_TILE + q * CHUNK, CHUNK)])
        return carry
    lax.fori_loop(0, NCHUNK, zbody, 0)

    wid = cid * NS + sid
    v0 = wid * SEGS_PER_TILE
    pltpu.sync_copy(bb_hbm.at[pl.ds(v0 * 64, SEGS_PER_TILE * 64)], bb_v)
    pltpu.sync_copy(gb_hbm, gb_v)
    pltpu.sync_copy(tok_hbm, tok_v)
    pltpu.sync_copy(mixv_hbm, mix_v)
    plsc.subcore_barrier()

    iot = lax.iota(jnp.int32, 16)

    def chunk_body(ci, carry):
        base = sid * PIX_PER_TILE + ci * CHUNK
        pltpu.sync_copy(tb_hbm.at[pl.ds(base, CHUNK)],
                        segblk_v.at[pl.ds(0, CHUNK)])
        for k in range(CHUNK // 16):
            s = segblk_v[pl.ds(k * 16, 16)] & 0xFFFF
            pp = iot + (base + k * 16)
            t = ((pp >> 16) << 4) + ((pp >> 6) & 15)
            idxh_v[k // 8, pl.ds((k % 8) * 16, 16)] = (s << 8) + t
        for q in range(16):
            pltpu.sync_copy(ones_v, grid_sh.at[idxh_v.at[q]], add=True)
        return carry
    lax.fori_loop(0, NCHUNK, chunk_body, 0)
    plsc.subcore_barrier()

    mix_b = mix_v[...]
    omix_b = 1.0 - mix_v[...]
    gbase = gb_v[pl.ds(0, 16)]

    def bbox(tloc):
        off = tloc * 64
        ymin = bb_v[pl.ds(off, 16)]
        xmin = bb_v[pl.ds(off + 16, 16)]
        ymax = bb_v[pl.ds(off + 32, 16)]
        xmax = bb_v[pl.ds(off + 48, 16)]
        return ymin, xmin, ymax, xmax

    def wrow(xmin, xmax):
        wpos = gbase * (xmax - xmin) + xmin
        wf = jnp.minimum(jnp.maximum(wpos.astype(jnp.int32), 0), W - 1)
        wc = jnp.minimum(wf + 1, W - 1)
        uw = wpos - wf.astype(jnp.float32)
        return wf, wc, uw

    def hrow(i, ymin, dh):
        g_i = gb_v[pl.ds(16 + i * 16, 16)]
        hpos = g_i * dh + ymin
        hf = jnp.minimum(jnp.maximum(hpos.astype(jnp.int32), 0), H - 1)
        hc = jnp.minimum(hf + 1, H - 1)
        uh = hpos - hf.astype(jnp.float32)
        return hf, hc, uh

    def build_idx(tloc):
        p = tloc & 1
        ymin, xmin, ymax, xmax = bbox(tloc)
        dh = ymax - ymin
        wf, wc, _ = wrow(xmin, xmax)

        def rows(prow_s):
            for i in range(16):
                hf, hc, _ = hrow(i, ymin, dh)
                rf_b = hf << 10
                rc_b = hc << 10
                for corner, idx in ((0, rf_b + wf), (1, rf_b + wc),
                                    (2, rc_b + wf), (3, rc_b + wc)):
                    e = corner * 256 + i * 16
                    gidx_v[prow_s + e // 128, pl.ds(e % 128, 16)] = idx

        @pl.when(p == 0)
        def _():
            rows(0)

        @pl.when(p == 1)
        def _():
            rows(8)

    def fire(tloc):
        p = tloc & 1
        prow = p * 8
        goff = p * 1024
        for q in range(8):
            pltpu.async_copy(ta_hbm.at[gidx_v.at[prow + q]],
                             ga_v.at[pl.ds(goff + q * 128, 128)], sem_g)
            pltpu.async_copy(tb_hbm.at[gidx_v.at[prow + q]],
                             gb2_v.at[pl.ds(goff + q * 128, 128)], sem_g)

    def g_drain():
        pltpu.make_async_copy(tb_hbm.at[pl.ds(0, 2048)],
                              segblk_v.at[pl.ds(0, 2048)], sem_g).wait()

    def out_drain():
        pltpu.make_async_copy(tb_hbm.at[pl.ds(0, 1024)],
                              outb_v, sem_sc).wait()

    def compute_seg(tloc):
        p = tloc & 1
        goff = p * 1024
        v = v0 + tloc

        @pl.when(tloc > 0)
        def _():
            out_drain()
        ymin, xmin, ymax, xmax = bbox(tloc)
        dh = ymax - ymin
        _, _, uw = wrow(xmin, xmax)
        lw = 1.0 - uw
        vi_b = jnp.broadcast_to(v, (16,))
        for i in range(16):
            _, _, uhb = hrow(i, ymin, dh)
            lhb = 1.0 - uhb
            wff = lhb * lw
            wfc = lhb * uw
            wcf = uhb * lw
            wcc = uhb * uw
            fs = []
            m = jnp.zeros((16,), jnp.float32)
            for corner, wgt in ((0, wff), (1, wfc), (2, wcf), (3, wcc)):
                base = goff + corner * 256 + i * 16
                wa = ga_v[pl.ds(base, 16)]
                wb = gb2_v[pl.ds(base, 16)]
                f0 = lax.bitcast_convert_type(wa & M_HI, jnp.float32)
                f1 = lax.bitcast_convert_type(wa << 16, jnp.float32)
                f2 = lax.bitcast_convert_type(wb & M_HI, jnp.float32)
                sg = wb & 0xFFFF
                fs.append((f0, f1, f2))
                m = m + jnp.where(sg == vi_b, wgt, 0.0)
            a_b = m + mix_b * (1.0 - m)
            t_b = omix_b * (1.0 - m)
            for c in range(3):
                bil = (fs[0][c] * wff + fs[1][c] * wfc
                       + fs[2][c] * wcf + fs[3][c] * wcc)
                tokci = tok_v[pl.ds(c * 256 + i * 16, 16)]
                outb_v[pl.ds(c * 256 + i * 16, 16)] = bil * a_b + t_b * tokci

        pltpu.sync_copy(grid_sh.at[pl.ds(v * 256, 256)], gr_v)
        acc = gr_v[pl.ds(0, 16)]
        for k in range(1, 16):
            acc = acc + gr_v[pl.ds(k * 16, 16)]
        for d in (8, 4, 2, 1):
            red_v[pl.ds(16, 16)] = acc
            plus = red_v[pl.ds(16 + d, 16)]
            minus = red_v[pl.ds(16 - d, 16)]
            acc = acc + jnp.where((iot & d) == 0, plus, minus)
        scale_b = 4.0 / acc
        for k in range(16):
            outb_v[pl.ds(768 + k * 16, 16)] = gr_v[pl.ds(k * 16, 16)] * scale_b
        pltpu.async_copy(outb_v, out_hbm.at[v], sem_sc)

    build_idx(jnp.int32(0))
    fire(jnp.int32(0))

    def seg_body(tloc, carry):
        build_idx(tloc)
        g_drain()
        fire(tloc)
        compute_seg(tloc - 1)
        return carry
    lax.fori_loop(1, SEGS_PER_TILE, seg_body, 0)
    g_drain()
    compute_seg(jnp.int32(SEGS_PER_TILE - 1))
    out_drain()


_sc_call = pl.kernel(
    _body,
    out_type=jax.ShapeDtypeStruct((NV, 4 * PS * PS), jnp.float32),
    mesh=plsc.VectorSubcoreMesh(core_axis_name="c", subcore_axis_name="s"),
    scratch_types=[
        pltpu.VMEM((CHUNK,), jnp.float32),
        pltpu.VMEM((128,), jnp.float32),
        pltpu.VMEM((2 * CHUNK,), jnp.int32),
        pltpu.VMEM((32, 128), jnp.int32),
        pltpu.VMEM((SEGS_PER_TILE * 64,), jnp.float32),
        pltpu.VMEM((16 + 256,), jnp.float32),
        pltpu.VMEM((3 * PS * PS,), jnp.float32),
        pltpu.VMEM((16,), jnp.float32),
        pltpu.VMEM((16, 128), jnp.int32),
        pltpu.VMEM((2048,), jnp.int32),
        pltpu.VMEM((2048,), jnp.int32),
        pltpu.VMEM((256,), jnp.float32),
        pltpu.VMEM((48,), jnp.float32),
        pltpu.VMEM((4 * PS * PS,), jnp.float32),
        pltpu.VMEM_SHARED((NV * PS * PS,), jnp.float32),
        pltpu.SemaphoreType.DMA,
        pltpu.SemaphoreType.DMA,
        pltpu.SemaphoreType.DMA,
    ],
)


def kernel(fV, seg, byx, bb, nV, pixel_mask_token, mix_logit):
    seg_u = seg.reshape(-1).astype(jnp.uint32)
    fb = fV.astype(jnp.bfloat16)
    u = lax.bitcast_convert_type(fb, jnp.uint16).astype(jnp.uint32)
    word_a = (u[:, 0] << 16) | u[:, 1]
    word_b = (u[:, 2] << 16) | seg_u
    tab_a = lax.bitcast_convert_type(word_a, jnp.int32)
    tab_b = lax.bitcast_convert_type(word_b, jnp.int32)
    bb_bc = jnp.repeat(bb.T.reshape(-1), 16)
    grid_base = jnp.linspace(0.0, 1.0, PS)
    gb_all = jnp.concatenate([grid_base, jnp.repeat(grid_base, 16)])
    mix = jax.nn.sigmoid(mix_logit)[0]
    mixv = jnp.full((16,), mix, jnp.float32)
    tok = pixel_mask_token.reshape(3 * PS * PS)
    out = _sc_call(tab_a, tab_b, bb_bc, gb_all, tok, mixv)
    return out.reshape(NV, 4, PS, PS)


# speedup vs baseline: 17.6945x; 1.0359x over previous
import jax
import jax.numpy as jnp
from jax import lax
from jax.experimental import pallas as pl
from jax.experimental.pallas import tpu as pltpu
from jax.experimental.pallas import tpu_sc as plsc

H = 1024
W = 1024
PS = 16
NV = 4096
N = H * W
NC = 2
NS = 16
PIX_PER_TILE = N // NS
CHUNK = 2048
NCHUNK = PIX_PER_TILE // CHUNK
SEGS_PER_TILE = NV // (NC * NS)
M_HI = -65536


def _body(ta_hbm, tb_hbm, bb_hbm, gb_hbm, tok_hbm, mixv_hbm, out_hbm,
          zeros_v, ones_v, segblk_v, idxh_v, bb_v, gb_v, tok_v, mix_v,
          gidx_v, ga_v, gb2_v, gr_v, red_v, outb_v, grid_sh,
          sem_g, sem_sc, sem_ld):
    sid = lax.axis_index("s")
    cid = lax.axis_index("c")

    zero16 = jnp.zeros((16,), jnp.float32)
    for k in range(128):
        zeros_v[pl.ds(k * 16, 16)] = zero16
    one16 = jnp.ones((16,), jnp.float32)
    for k in range(8):
        ones_v[pl.ds(k * 16, 16)] = one16
    red_v[pl.ds(0, 16)] = zero16
    red_v[pl.ds(32, 16)] = zero16

    def zbody(q, carry):
        pltpu.sync_copy(zeros_v,
                        grid_sh.at[pl.ds(sid * PIX_PER_TILE + q * CHUNK, CHUNK)])
        return carry
    lax.fori_loop(0, NCHUNK, zbody, 0)

    wid = cid * NS + sid
    v0 = wid * SEGS_PER_TILE
    pltpu.sync_copy(bb_hbm.at[pl.ds(v0 * 64, SEGS_PER_TILE * 64)], bb_v)
    pltpu.sync_copy(gb_hbm, gb_v)
    pltpu.sync_copy(tok_hbm, tok_v)
    pltpu.sync_copy(mixv_hbm, mix_v)
    plsc.subcore_barrier()

    iot = lax.iota(jnp.int32, 16)
    pix0 = sid * PIX_PER_TILE
    pltpu.sync_copy(tb_hbm.at[pl.ds(pix0, CHUNK)],
                    segblk_v.at[pl.ds(0, CHUNK)])

    def chunk_body(ci, carry):
        p = ci & 1
        base = pix0 + ci * CHUNK

        @pl.when(ci < NCHUNK - 1)
        def _():
            pltpu.async_copy(
                tb_hbm.at[pl.ds(base + CHUNK, CHUNK)],
                segblk_v.at[pl.ds((1 - p) * CHUNK, CHUNK)], sem_ld)

        def do_chunk(prow_s, pblk_s):
            for k in range(CHUNK // 16):
                s = segblk_v[pl.ds(pblk_s + k * 16, 16)] & 0xFFFF
                pp = iot + (base + k * 16)
                t = ((pp >> 16) << 4) + ((pp >> 6) & 15)
                idxh_v[prow_s + k // 8, pl.ds((k % 8) * 16, 16)] = (s << 8) + t
            for q in range(16):
                pltpu.sync_copy(ones_v, grid_sh.at[idxh_v.at[prow_s + q]],
                                add=True)

        @pl.when(p == 0)
        def _():
            do_chunk(0, 0)

        @pl.when(p == 1)
        def _():
            do_chunk(16, CHUNK)

        @pl.when(ci < NCHUNK - 1)
        def _():
            pltpu.make_async_copy(tb_hbm.at[pl.ds(0, CHUNK)],
                                  segblk_v.at[pl.ds(0, CHUNK)],
                                  sem_ld).wait()
        return carry
    lax.fori_loop(0, NCHUNK, chunk_body, 0)
    plsc.subcore_barrier()

    mix_b = mix_v[...]
    omix_b = 1.0 - mix_v[...]
    gbase = gb_v[pl.ds(0, 16)]

    def bbox(tloc):
        off = tloc * 64
        ymin = bb_v[pl.ds(off, 16)]
        xmin = bb_v[pl.ds(off + 16, 16)]
        ymax = bb_v[pl.ds(off + 32, 16)]
        xmax = bb_v[pl.ds(off + 48, 16)]
        return ymin, xmin, ymax, xmax

    def wrow(xmin, xmax):
        wpos = gbase * (xmax - xmin) + xmin
        wf = jnp.minimum(jnp.maximum(wpos.astype(jnp.int32), 0), W - 1)
        wc = jnp.minimum(wf + 1, W - 1)
        uw = wpos - wf.astype(jnp.float32)
        return wf, wc, uw

    def hrow(i, ymin, dh):
        g_i = gb_v[pl.ds(16 + i * 16, 16)]
        hpos = g_i * dh + ymin
        hf = jnp.minimum(jnp.maximum(hpos.astype(jnp.int32), 0), H - 1)
        hc = jnp.minimum(hf + 1, H - 1)
        uh = hpos - hf.astype(jnp.float32)
        return hf, hc, uh

    def build_idx(tloc):
        p = tloc & 1
        ymin, xmin, ymax, xmax = bbox(tloc)
        dh = ymax - ymin
        wf, wc, _ = wrow(xmin, xmax)

        def rows(prow_s):
            for i in range(16):
                hf, hc, _ = hrow(i, ymin, dh)
                rf_b = hf << 10
                rc_b = hc << 10
                for corner, idx in ((0, rf_b + wf), (1, rf_b + wc),
                                    (2, rc_b + wf), (3, rc_b + wc)):
                    e = corner * 256 + i * 16
                    gidx_v[prow_s + e // 128, pl.ds(e % 128, 16)] = idx

        @pl.when(p == 0)
        def _():
            rows(0)

        @pl.when(p == 1)
        def _():
            rows(8)

    def fire(tloc):
        p = tloc & 1
        prow = p * 8
        goff = p * 1024
        for q in range(8):
            pltpu.async_copy(ta_hbm.at[gidx_v.at[prow + q]],
                             ga_v.at[pl.ds(goff + q * 128, 128)], sem_g)
            pltpu.async_copy(tb_hbm.at[gidx_v.at[prow + q]],
                             gb2_v.at[pl.ds(goff + q * 128, 128)], sem_g)

    def g_drain():
        pltpu.make_async_copy(tb_hbm.at[pl.ds(0, 2048)],
                              segblk_v.at[pl.ds(0, 2048)], sem_g).wait()

    def out_drain():
        pltpu.make_async_copy(tb_hbm.at[pl.ds(0, 1024)],
                              outb_v, sem_sc).wait()

    def compute_seg(tloc):
        p = tloc & 1
        goff = p * 1024
        v = v0 + tloc

        @pl.when(tloc > 0)
        def _():
            out_drain()
        ymin, xmin, ymax, xmax = bbox(tloc)
        dh = ymax - ymin
        _, _, uw = wrow(xmin, xmax)
        lw = 1.0 - uw
        vi_b = jnp.broadcast_to(v, (16,))
        for i in range(16):
            _, _, uhb = hrow(i, ymin, dh)
            lhb = 1.0 - uhb
            wff = lhb * lw
            wfc = lhb * uw
            wcf = uhb * lw
            wcc = uhb * uw
            fs = []
            m = jnp.zeros((16,), jnp.float32)
            for corner, wgt in ((0, wff), (1, wfc), (2, wcf), (3, wcc)):
                base = goff + corner * 256 + i * 16
                wa = ga_v[pl.ds(base, 16)]
                wb = gb2_v[pl.ds(base, 16)]
                f0 = lax.bitcast_convert_type(wa & M_HI, jnp.float32)
                f1 = lax.bitcast_convert_type(wa << 16, jnp.float32)
                f2 = lax.bitcast_convert_type(wb & M_HI, jnp.float32)
                sg = wb & 0xFFFF
                fs.append((f0, f1, f2))
                m = m + jnp.where(sg == vi_b, wgt, 0.0)
            a_b = m + mix_b * (1.0 - m)
            t_b = omix_b * (1.0 - m)
            for c in range(3):
                bil = (fs[0][c] * wff + fs[1][c] * wfc
                       + fs[2][c] * wcf + fs[3][c] * wcc)
                tokci = tok_v[pl.ds(c * 256 + i * 16, 16)]
                outb_v[pl.ds(c * 256 + i * 16, 16)] = bil * a_b + t_b * tokci

        pltpu.sync_copy(grid_sh.at[pl.ds(v * 256, 256)], gr_v)
        acc = gr_v[pl.ds(0, 16)]
        for k in range(1, 16):
            acc = acc + gr_v[pl.ds(k * 16, 16)]
        for d in (8, 4, 2, 1):
            red_v[pl.ds(16, 16)] = acc
            plus = red_v[pl.ds(16 + d, 16)]
            minus = red_v[pl.ds(16 - d, 16)]
            acc = acc + jnp.where((iot & d) == 0, plus, minus)
        scale_b = 4.0 / acc
        for k in range(16):
            outb_v[pl.ds(768 + k * 16, 16)] = gr_v[pl.ds(k * 16, 16)] * scale_b
        pltpu.async_copy(outb_v, out_hbm.at[v], sem_sc)

    build_idx(jnp.int32(0))
    fire(jnp.int32(0))

    def seg_body(tloc, carry):
        build_idx(tloc)
        g_drain()
        fire(tloc)
        compute_seg(tloc - 1)
        return carry
    lax.fori_loop(1, SEGS_PER_TILE, seg_body, 0)
    g_drain()
    compute_seg(jnp.int32(SEGS_PER_TILE - 1))
    out_drain()


_sc_call = pl.kernel(
    _body,
    out_type=jax.ShapeDtypeStruct((NV, 4 * PS * PS), jnp.float32),
    mesh=plsc.VectorSubcoreMesh(core_axis_name="c", subcore_axis_name="s"),
    scratch_types=[
        pltpu.VMEM((CHUNK,), jnp.float32),
        pltpu.VMEM((128,), jnp.float32),
        pltpu.VMEM((2 * CHUNK,), jnp.int32),
        pltpu.VMEM((32, 128), jnp.int32),
        pltpu.VMEM((SEGS_PER_TILE * 64,), jnp.float32),
        pltpu.VMEM((16 + 256,), jnp.float32),
        pltpu.VMEM((3 * PS * PS,), jnp.float32),
        pltpu.VMEM((16,), jnp.float32),
        pltpu.VMEM((16, 128), jnp.int32),
        pltpu.VMEM((2048,), jnp.int32),
        pltpu.VMEM((2048,), jnp.int32),
        pltpu.VMEM((256,), jnp.float32),
        pltpu.VMEM((48,), jnp.float32),
        pltpu.VMEM((4 * PS * PS,), jnp.float32),
        pltpu.VMEM_SHARED((NV * PS * PS,), jnp.float32),
        pltpu.SemaphoreType.DMA,
        pltpu.SemaphoreType.DMA,
        pltpu.SemaphoreType.DMA,
    ],
)


def kernel(fV, seg, byx, bb, nV, pixel_mask_token, mix_logit):
    seg_u = seg.reshape(-1).astype(jnp.uint32)
    fb = fV.astype(jnp.bfloat16)
    u = lax.bitcast_convert_type(fb, jnp.uint16).astype(jnp.uint32)
    word_a = (u[:, 0] << 16) | u[:, 1]
    word_b = (u[:, 2] << 16) | seg_u
    tab_a = lax.bitcast_convert_type(word_a, jnp.int32)
    tab_b = lax.bitcast_convert_type(word_b, jnp.int32)
    bb_bc = jnp.repeat(bb.T.reshape(-1), 16)
    grid_base = jnp.linspace(0.0, 1.0, PS)
    gb_all = jnp.concatenate([grid_base, jnp.repeat(grid_base, 16)])
    mix = jax.nn.sigmoid(mix_logit)[0]
    mixv = jnp.full((16,), mix, jnp.float32)
    tok = pixel_mask_token.reshape(3 * PS * PS)
    out = _sc_call(tab_a, tab_b, bb_bc, gb_all, tok, mixv)
    return out.reshape(NV, 4, PS, PS)
